# Initial kernel scaffold; baseline (speedup 1.0000x reference)
import jax
import jax.numpy as jnp
from jax.experimental import pallas as pl


def kernel(nodes, senders, receivers, n_node, W_embed, b_embed, W_mlp, b_mlp, ln_scale, ln_bias, W_dec, b_dec):
    raise NotImplementedError("write your pallas kernel here")


# speedup vs baseline: 8.2697x; 8.2697x over previous
import functools

import jax
import jax.numpy as jnp
from jax import lax
from jax.experimental import pallas as pl
from jax.experimental.pallas import tpu as pltpu
from jax.experimental.pallas import tpu_sc as plsc

N = 10000
E = 320000
D = 128
G = 16
STEPS = 2
NPAD = 10240
BLK = 2000
PER_G = N // G

SC_CORES = 2
SC_TILES = 16
NW = SC_CORES * SC_TILES
ROWS_PER_TILE = NPAD // SC_TILES
UNIT = 128
NUNITS = E // UNIT
UNITS_PER_W = (NUNITS + NW - 1) // NW

_sc_mesh = plsc.VectorSubcoreMesh(core_axis_name="c", subcore_axis_name="s")


def _embed_body(nodes_ref, w_ref, b_ref, o_ref):
    o_ref[...] = (
        jnp.dot(nodes_ref[...], w_ref[...], preferred_element_type=jnp.float32)
        + b_ref[...]
    )


def _tc_embed(nodes, w, b):
    return pl.pallas_call(
        _embed_body,
        grid=(N // BLK,),
        in_specs=[
            pl.BlockSpec((BLK, D), lambda i: (i, 0)),
            pl.BlockSpec((D, D), lambda i: (0, 0)),
            pl.BlockSpec((1, D), lambda i: (0, 0)),
        ],
        out_specs=pl.BlockSpec((BLK, D), lambda i: (i, 0)),
        out_shape=jax.ShapeDtypeStruct((N, D), jnp.float32),
    )(nodes, w, b.reshape(1, D))


def _inv_body(degp_ref, o_ref):
    d = degp_ref[0] + degp_ref[1] + 1.0
    o_ref[...] = lax.rsqrt(jnp.maximum(d, 1.0))


def _tc_inv(degp):
    return pl.pallas_call(
        _inv_body,
        out_shape=jax.ShapeDtypeStruct((2, NPAD), jnp.float32),
    )(degp)


def _mlp_body(x_ref, w_ref, b_ref, inv_ref, o_ref):
    h = jnp.dot(x_ref[...], w_ref[...], preferred_element_type=jnp.float32)
    h = jnp.maximum(h + b_ref[...], 0.0)
    o_ref[...] = h * inv_ref[...]


def _tc_mlp(x, w, b, inv_s_col):
    return pl.pallas_call(
        _mlp_body,
        grid=(N // BLK,),
        in_specs=[
            pl.BlockSpec((BLK, D), lambda i: (i, 0)),
            pl.BlockSpec((D, D), lambda i: (0, 0)),
            pl.BlockSpec((1, D), lambda i: (0, 0)),
            pl.BlockSpec((BLK, 1), lambda i: (i, 0)),
        ],
        out_specs=pl.BlockSpec((BLK, D), lambda i: (i, 0)),
        out_shape=jax.ShapeDtypeStruct((N, D), jnp.float32),
    )(x, w, b.reshape(1, D), inv_s_col)


def _post_body(aggp_ref, h_ref, x_ref, inv_ref, s_ref, b_ref, o_ref):
    agg = aggp_ref[0] + aggp_ref[1] + h_ref[...]
    t = agg * inv_ref[...] + x_ref[...]
    mu = jnp.mean(t, axis=-1, keepdims=True)
    var = jnp.mean(jnp.square(t - mu), axis=-1, keepdims=True)
    o_ref[...] = (t - mu) * lax.rsqrt(var + 1e-6) * s_ref[...] + b_ref[...]


def _tc_post(aggp, h, x, inv_r_col, scale, bias):
    return pl.pallas_call(
        _post_body,
        grid=(N // BLK,),
        in_specs=[
            pl.BlockSpec((2, BLK, D), lambda i: (0, i, 0)),
            pl.BlockSpec((BLK, D), lambda i: (i, 0)),
            pl.BlockSpec((BLK, D), lambda i: (i, 0)),
            pl.BlockSpec((BLK, 1), lambda i: (i, 0)),
            pl.BlockSpec((1, D), lambda i: (0, 0)),
            pl.BlockSpec((1, D), lambda i: (0, 0)),
        ],
        out_specs=pl.BlockSpec((BLK, D), lambda i: (i, 0)),
        out_shape=jax.ShapeDtypeStruct((N, D), jnp.float32),
    )(aggp, h, x, inv_r_col, scale.reshape(1, D), bias.reshape(1, D))


def _pool_body(x_ref, p_ref, cnt_ref, w_ref, b_ref, o_ref):
    pooled = lax.dot_general(
        p_ref[...], x_ref[...], (((0,), (0,)), ((), ())),
        preferred_element_type=jnp.float32,
    )
    pooled = pooled / cnt_ref[...]
    o_ref[...] = (
        jnp.dot(pooled, w_ref[...], preferred_element_type=jnp.float32)
        + b_ref[...]
    )


def _tc_pool_decode(x, n_node, w_dec, b_dec):
    bounds = jnp.cumsum(n_node)
    node_graph = jnp.sum(
        jnp.arange(N, dtype=jnp.int32)[:, None] >= bounds[None, :], axis=1
    )
    p = (node_graph[:, None] == jnp.arange(G, dtype=jnp.int32)[None, :])
    p = p.astype(jnp.float32)
    counts = jnp.maximum(n_node.astype(jnp.float32), 1.0).reshape(G, 1)
    return pl.pallas_call(
        _pool_body,
        out_shape=jax.ShapeDtypeStruct((G, D), jnp.float32),
    )(x, p, counts, w_dec, b_dec.reshape(1, D))


@functools.partial(
    pl.kernel,
    out_type=jax.ShapeDtypeStruct((SC_CORES, 2, NPAD), jnp.float32),
    mesh=_sc_mesh,
    scratch_types=[
        pltpu.VMEM((1, UNIT), jnp.int32),
        pltpu.VMEM((1, UNIT), jnp.int32),
        pltpu.VMEM((UNIT,), jnp.float32),
        pltpu.VMEM((ROWS_PER_TILE,), jnp.float32),
        pltpu.VMEM_SHARED((NPAD,), jnp.float32),
        pltpu.VMEM_SHARED((NPAD,), jnp.float32),
    ],
)
def _sc_degrees(send_hbm, recv_hbm, out_hbm, sidx, ridx, ones_v, tmp,
                acc_s, acc_r):
    cid = lax.axis_index("c")
    sid = lax.axis_index("s")
    wid = sid * SC_CORES + cid
    for k in range(UNIT // 16):
        ones_v[pl.ds(k * 16, 16)] = jnp.ones((16,), jnp.float32)
    for k in range(ROWS_PER_TILE // 16):
        tmp[pl.ds(k * 16, 16)] = jnp.zeros((16,), jnp.float32)
    base_r = sid * ROWS_PER_TILE
    pltpu.sync_copy(tmp, acc_s.at[pl.ds(base_r, ROWS_PER_TILE)])
    pltpu.sync_copy(tmp, acc_r.at[pl.ds(base_r, ROWS_PER_TILE)])
    plsc.subcore_barrier()

    def body(j, carry):
        u = wid + NW * j

        @pl.when(u < NUNITS)
        def _():
            pltpu.sync_copy(send_hbm.at[pl.ds(u * UNIT, UNIT)], sidx.at[0])
            pltpu.sync_copy(recv_hbm.at[pl.ds(u * UNIT, UNIT)], ridx.at[0])
            pltpu.sync_copy(ones_v, acc_s.at[sidx.at[0]], add=True)
            pltpu.sync_copy(ones_v, acc_r.at[ridx.at[0]], add=True)

        return carry

    lax.fori_loop(0, UNITS_PER_W, body, 0)
    plsc.subcore_barrier()
    pltpu.sync_copy(acc_s.at[pl.ds(base_r, ROWS_PER_TILE)], tmp)
    pltpu.sync_copy(tmp, out_hbm.at[cid, 0, pl.ds(base_r, ROWS_PER_TILE)])
    pltpu.sync_copy(acc_r.at[pl.ds(base_r, ROWS_PER_TILE)], tmp)
    pltpu.sync_copy(tmp, out_hbm.at[cid, 1, pl.ds(base_r, ROWS_PER_TILE)])


@functools.partial(
    pl.kernel,
    out_type=jax.ShapeDtypeStruct((SC_CORES, NPAD, D), jnp.float32),
    mesh=_sc_mesh,
    scratch_types=[
        pltpu.VMEM((UNIT,), jnp.int32),
        pltpu.VMEM((1, UNIT), jnp.int32),
        pltpu.VMEM((UNIT, D), jnp.float32),
        pltpu.VMEM((UNIT, D), jnp.float32),
        pltpu.VMEM_SHARED((NPAD, D), jnp.float32),
        pltpu.SemaphoreType.DMA,
    ],
)
def _sc_edge_agg(h_hbm, send_hbm, recv_hbm, out_hbm, sidx, ridx, rows, zbuf,
                 acc, sem):
    cid = lax.axis_index("c")
    sid = lax.axis_index("s")
    wid = sid * SC_CORES + cid

    def zrow(r, carry):
        for k in range(D // 16):
            zbuf[r, pl.ds(k * 16, 16)] = jnp.zeros((16,), jnp.float32)
        return carry

    lax.fori_loop(0, UNIT, zrow, 0)
    base_r = sid * ROWS_PER_TILE
    for k in range(ROWS_PER_TILE // UNIT):
        pltpu.sync_copy(zbuf, acc.at[pl.ds(base_r + k * UNIT, UNIT)])
    plsc.subcore_barrier()

    def body(j, carry):
        u = wid + NW * j

        @pl.when(u < NUNITS)
        def _():
            pltpu.sync_copy(send_hbm.at[pl.ds(u * UNIT, UNIT)], sidx)
            pltpu.sync_copy(recv_hbm.at[pl.ds(u * UNIT, UNIT)], ridx.at[0])
            pltpu.async_copy(h_hbm.at[sidx], rows, sem).wait()
            pltpu.sync_copy(rows, acc.at[ridx.at[0]], add=True)

        return carry

    lax.fori_loop(0, UNITS_PER_W, body, 0)
    plsc.subcore_barrier()
    for k in range(ROWS_PER_TILE // UNIT):
        r0 = base_r + k * UNIT
        pltpu.sync_copy(acc.at[pl.ds(r0, UNIT)], zbuf)
        pltpu.sync_copy(zbuf, out_hbm.at[cid, pl.ds(r0, UNIT)])


def kernel(nodes, senders, receivers, n_node, W_embed, b_embed, W_mlp, b_mlp,
           ln_scale, ln_bias, W_dec, b_dec):
    degp = _sc_degrees(senders, receivers)
    inv = _tc_inv(degp)
    inv_s_col = inv[0].reshape(NPAD, 1)
    inv_r_col = inv[1].reshape(NPAD, 1)
    x = _tc_embed(nodes, W_embed, b_embed)
    for s in range(STEPS):
        h = _tc_mlp(x, W_mlp[s], b_mlp[s], inv_s_col)
        aggp = _sc_edge_agg(h, senders, receivers)
        x = _tc_post(aggp, h, x, inv_r_col, ln_scale[s], ln_bias[s])
    return _tc_pool_decode(x, n_node, W_dec, b_dec)


# speedup vs baseline: 17.2895x; 2.0907x over previous
import functools

import jax
import jax.numpy as jnp
from jax import lax
from jax.experimental import pallas as pl
from jax.experimental.pallas import tpu as pltpu
from jax.experimental.pallas import tpu_sc as plsc

N = 10000
E = 320000
D = 128
G = 16
STEPS = 2
NPAD = 10240
BLK = 2000

SC_CORES = 2
SC_TILES = 16
NW = SC_CORES * SC_TILES
ROWS_PER_TILE = NPAD // SC_TILES
UNIT = 128
NUNITS = E // UNIT
NU = NUNITS // NW
NEXTRA = NUNITS - NU * NW
WIN = 40

_sc_mesh = plsc.VectorSubcoreMesh(core_axis_name="c", subcore_axis_name="s")


def _embed_body(nodes_ref, w_ref, b_ref, o_ref):
    o_ref[...] = (
        jnp.dot(nodes_ref[...], w_ref[...], preferred_element_type=jnp.float32)
        + b_ref[...]
    )


def _tc_embed(nodes, w, b):
    return pl.pallas_call(
        _embed_body,
        grid=(N // BLK,),
        in_specs=[
            pl.BlockSpec((BLK, D), lambda i: (i, 0)),
            pl.BlockSpec((D, D), lambda i: (0, 0)),
            pl.BlockSpec((1, D), lambda i: (0, 0)),
        ],
        out_specs=pl.BlockSpec((BLK, D), lambda i: (i, 0)),
        out_shape=jax.ShapeDtypeStruct((N, D), jnp.float32),
    )(nodes, w, b.reshape(1, D))


def _inv_body(degp_ref, o_ref):
    d = degp_ref[0] + degp_ref[1] + 1.0
    o_ref[...] = lax.rsqrt(jnp.maximum(d, 1.0))


def _tc_inv(degp):
    return pl.pallas_call(
        _inv_body,
        out_shape=jax.ShapeDtypeStruct((2, NPAD), jnp.float32),
    )(degp)


def _mlp_body(x_ref, w_ref, b_ref, inv_ref, o_ref):
    h = jnp.dot(x_ref[...], w_ref[...], preferred_element_type=jnp.float32)
    h = jnp.maximum(h + b_ref[...], 0.0)
    o_ref[...] = h * inv_ref[...]


def _tc_mlp(x, w, b, inv_s_col):
    return pl.pallas_call(
        _mlp_body,
        grid=(N // BLK,),
        in_specs=[
            pl.BlockSpec((BLK, D), lambda i: (i, 0)),
            pl.BlockSpec((D, D), lambda i: (0, 0)),
            pl.BlockSpec((1, D), lambda i: (0, 0)),
            pl.BlockSpec((BLK, 1), lambda i: (i, 0)),
        ],
        out_specs=pl.BlockSpec((BLK, D), lambda i: (i, 0)),
        out_shape=jax.ShapeDtypeStruct((N, D), jnp.float32),
    )(x, w, b.reshape(1, D), inv_s_col)


def _post_body(aggp_ref, h_ref, x_ref, inv_ref, s_ref, b_ref, o_ref):
    agg = aggp_ref[0] + aggp_ref[1] + h_ref[...]
    t = agg * inv_ref[...] + x_ref[...]
    mu = jnp.mean(t, axis=-1, keepdims=True)
    var = jnp.mean(jnp.square(t - mu), axis=-1, keepdims=True)
    o_ref[...] = (t - mu) * lax.rsqrt(var + 1e-6) * s_ref[...] + b_ref[...]


def _tc_post(aggp, h, x, inv_r_col, scale, bias):
    return pl.pallas_call(
        _post_body,
        grid=(N // BLK,),
        in_specs=[
            pl.BlockSpec((2, BLK, D), lambda i: (0, i, 0)),
            pl.BlockSpec((BLK, D), lambda i: (i, 0)),
            pl.BlockSpec((BLK, D), lambda i: (i, 0)),
            pl.BlockSpec((BLK, 1), lambda i: (i, 0)),
            pl.BlockSpec((1, D), lambda i: (0, 0)),
            pl.BlockSpec((1, D), lambda i: (0, 0)),
        ],
        out_specs=pl.BlockSpec((BLK, D), lambda i: (i, 0)),
        out_shape=jax.ShapeDtypeStruct((N, D), jnp.float32),
    )(aggp, h, x, inv_r_col, scale.reshape(1, D), bias.reshape(1, D))


def _pool_body(x_ref, p_ref, cnt_ref, w_ref, b_ref, o_ref):
    pooled = lax.dot_general(
        p_ref[...], x_ref[...], (((0,), (0,)), ((), ())),
        preferred_element_type=jnp.float32,
    )
    pooled = pooled / cnt_ref[...]
    o_ref[...] = (
        jnp.dot(pooled, w_ref[...], preferred_element_type=jnp.float32)
        + b_ref[...]
    )


def _tc_pool_decode(x, n_node, w_dec, b_dec):
    bounds = jnp.cumsum(n_node)
    node_graph = jnp.sum(
        jnp.arange(N, dtype=jnp.int32)[:, None] >= bounds[None, :], axis=1
    )
    p = (node_graph[:, None] == jnp.arange(G, dtype=jnp.int32)[None, :])
    p = p.astype(jnp.float32)
    counts = jnp.maximum(n_node.astype(jnp.float32), 1.0).reshape(G, 1)
    return pl.pallas_call(
        _pool_body,
        out_shape=jax.ShapeDtypeStruct((G, D), jnp.float32),
    )(x, p, counts, w_dec, b_dec.reshape(1, D))


def _unit_range(wid):
    start = NU * wid + jnp.maximum(wid - (NW - NEXTRA), 0)
    cnt = NU + (wid >= (NW - NEXTRA)).astype(jnp.int32)
    return start, cnt


@functools.partial(
    pl.kernel,
    out_type=jax.ShapeDtypeStruct((SC_CORES, 2, NPAD), jnp.float32),
    mesh=_sc_mesh,
    scratch_types=[
        pltpu.VMEM((NU + 1, 2, UNIT), jnp.int32),
        pltpu.VMEM((UNIT,), jnp.float32),
        pltpu.VMEM((ROWS_PER_TILE,), jnp.float32),
        pltpu.VMEM_SHARED((NPAD,), jnp.float32),
        pltpu.VMEM_SHARED((NPAD,), jnp.float32),
    ],
)
def _sc_degrees(comb_hbm, out_hbm, cbuf, ones_v, tmp, acc_s, acc_r):
    cid = lax.axis_index("c")
    sid = lax.axis_index("s")
    wid = sid * SC_CORES + cid
    start, cnt = _unit_range(wid)
    pltpu.sync_copy(comb_hbm.at[pl.ds(start, NU)], cbuf.at[pl.ds(0, NU)])

    @pl.when(cnt > NU)
    def _():
        pltpu.sync_copy(comb_hbm.at[pl.ds(start + NU, 1)],
                        cbuf.at[pl.ds(NU, 1)])

    for k in range(UNIT // 16):
        ones_v[pl.ds(k * 16, 16)] = jnp.ones((16,), jnp.float32)
    for k in range(ROWS_PER_TILE // 16):
        tmp[pl.ds(k * 16, 16)] = jnp.zeros((16,), jnp.float32)
    base_r = sid * ROWS_PER_TILE
    pltpu.sync_copy(tmp, acc_s.at[pl.ds(base_r, ROWS_PER_TILE)])
    pltpu.sync_copy(tmp, acc_r.at[pl.ds(base_r, ROWS_PER_TILE)])
    plsc.subcore_barrier()

    def body(j, carry):
        pltpu.sync_copy(ones_v, acc_s.at[cbuf.at[j, 0]], add=True)
        pltpu.sync_copy(ones_v, acc_r.at[cbuf.at[j, 1]], add=True)
        return carry

    lax.fori_loop(0, cnt, body, 0)
    plsc.subcore_barrier()
    pltpu.sync_copy(acc_s.at[pl.ds(base_r, ROWS_PER_TILE)], tmp)
    pltpu.sync_copy(tmp, out_hbm.at[cid, 0, pl.ds(base_r, ROWS_PER_TILE)])
    pltpu.sync_copy(acc_r.at[pl.ds(base_r, ROWS_PER_TILE)], tmp)
    pltpu.sync_copy(tmp, out_hbm.at[cid, 1, pl.ds(base_r, ROWS_PER_TILE)])


@functools.partial(
    pl.kernel,
    out_type=jax.ShapeDtypeStruct((SC_CORES, NPAD, D), jnp.float32),
    mesh=_sc_mesh,
    scratch_types=[
        pltpu.VMEM((WIN, 2, UNIT), jnp.int32),
        pltpu.VMEM((UNIT, D), jnp.float32),
        pltpu.VMEM((UNIT, D), jnp.float32),
        pltpu.VMEM_SHARED((NPAD, D), jnp.float32),
        pltpu.SemaphoreType.DMA,
        pltpu.SemaphoreType.DMA,
    ],
)
def _sc_edge_agg(h_hbm, comb_hbm, out_hbm, cbuf, rows0, rows1, acc,
                 sem0, sem1):
    cid = lax.axis_index("c")
    sid = lax.axis_index("s")
    wid = sid * SC_CORES + cid
    start, cnt = _unit_range(wid)

    def zrow(r, carry):
        for k in range(D // 16):
            rows0[r, pl.ds(k * 16, 16)] = jnp.zeros((16,), jnp.float32)
        return carry

    lax.fori_loop(0, UNIT, zrow, 0)
    base_r = sid * ROWS_PER_TILE
    for k in range(ROWS_PER_TILE // UNIT):
        pltpu.sync_copy(rows0, acc.at[pl.ds(base_r + k * UNIT, UNIT)])
    plsc.subcore_barrier()

    def window(w0, wcnt):
        pltpu.sync_copy(comb_hbm.at[pl.ds(w0, WIN)], cbuf)
        pltpu.async_copy(h_hbm.at[cbuf.at[0, 0]], rows0, sem0)

        def body(p, carry):
            u0 = 2 * p

            @pl.when(u0 + 1 < wcnt)
            def _():
                pltpu.async_copy(h_hbm.at[cbuf.at[u0 + 1, 0]], rows1, sem1)

            pltpu.make_async_copy(h_hbm.at[cbuf.at[u0, 0]], rows0,
                                  sem0).wait()
            pltpu.sync_copy(rows0, acc.at[cbuf.at[u0, 1]], add=True)

            @pl.when(u0 + 2 < wcnt)
            def _():
                pltpu.async_copy(h_hbm.at[cbuf.at[u0 + 2, 0]], rows0, sem0)

            @pl.when(u0 + 1 < wcnt)
            def _():
                pltpu.make_async_copy(h_hbm.at[cbuf.at[u0 + 1, 0]], rows1,
                                      sem1).wait()
                pltpu.sync_copy(rows1, acc.at[cbuf.at[u0 + 1, 1]], add=True)

            return carry

        lax.fori_loop(0, (wcnt + 1) // 2, body, 0)

    window(start, jnp.int32(WIN))
    window(start + WIN, cnt - WIN)

    plsc.subcore_barrier()
    for k in range(ROWS_PER_TILE // UNIT):
        r0 = base_r + k * UNIT
        pltpu.sync_copy(acc.at[pl.ds(r0, UNIT)], rows0)
        pltpu.sync_copy(rows0, out_hbm.at[cid, pl.ds(r0, UNIT)])


def kernel(nodes, senders, receivers, n_node, W_embed, b_embed, W_mlp, b_mlp,
           ln_scale, ln_bias, W_dec, b_dec):
    comb = jnp.stack(
        [senders.reshape(NUNITS, UNIT), receivers.reshape(NUNITS, UNIT)],
        axis=1,
    )
    comb = jnp.pad(comb, ((0, 4), (0, 0), (0, 0)))
    degp = _sc_degrees(comb)
    inv = _tc_inv(degp)
    inv_s_col = inv[0].reshape(NPAD, 1)
    inv_r_col = inv[1].reshape(NPAD, 1)
    x = _tc_embed(nodes, W_embed, b_embed)
    for s in range(STEPS):
        h = _tc_mlp(x, W_mlp[s], b_mlp[s], inv_s_col)
        aggp = _sc_edge_agg(h, comb)
        x = _tc_post(aggp, h, x, inv_r_col, ln_scale[s], ln_bias[s])
    return _tc_pool_decode(x, n_node, W_dec, b_dec)
